# baseline (speedup 1.0000x reference)
import functools
import math

import jax
import jax.numpy as jnp
from jax import lax
from jax.experimental import pallas as pl
from jax.experimental.pallas import tpu as pltpu
from jax.experimental.pallas import tpu_sc as plsc

B, N, K = 2, 4096, 16
DP, D = 64, 128
CP = 16
RB = 256
RC = 256
RP = 512
W = 256

SPB = 2
SEG = B * SPB
SN = N // SPB
SROWS = SN * K

NC, NS = 2, 16
NW = NC * NS
RPW = SROWS // NW
CH = 128
NCH = RPW // CH


def _prep_body(feat_ref, z_ref, w_ref, b_ref, out_ref):
    x = (
        jnp.dot(feat_ref[...], w_ref[...], preferred_element_type=jnp.float32)
        + b_ref[...]
    )
    out_ref[...] = jnp.concatenate(
        [x, z_ref[...], jnp.zeros((RP, W - D - CP), jnp.float32)], axis=1)


_NET8 = [(0, 1), (2, 3), (4, 5), (6, 7),
         (0, 2), (1, 3), (4, 6), (5, 7),
         (1, 2), (5, 6),
         (0, 4), (1, 5), (2, 6), (3, 7),
         (2, 4), (3, 5),
         (1, 2), (3, 4), (5, 6)]
G = 8
GW = N // G


def _knn_body(xyzp_ref, xyzt_ref, out_ref, *, offs):
    blk = xyzp_ref[...]
    allt = xyzt_ref[...]
    sq_src = jnp.sum(blk * blk, axis=1, keepdims=True)
    sq_dst = jnp.sum(allt * allt, axis=0, keepdims=True)
    dot = jnp.dot(blk, allt, preferred_element_type=jnp.float32)
    dist = sq_src - 2.0 * dot + sq_dst
    fiota = lax.broadcasted_iota(jnp.int32, (RB, N), 1).astype(jnp.float32)
    kiota = lax.broadcasted_iota(jnp.int32, (RB, K), 1)
    big = jnp.float32(2.0 * N)
    inf = jnp.float32(jnp.inf)

    vs = [dist[:, j * GW:(j + 1) * GW] for j in range(G)]
    ix = [fiota[:, j * GW:(j + 1) * GW] for j in range(G)]
    for a, b in _NET8:
        va, vb, ia, ib = vs[a], vs[b], ix[a], ix[b]
        c = (va < vb) | ((va == vb) & (ia < ib))
        vs[a] = jnp.where(c, va, vb)
        vs[b] = jnp.where(c, vb, va)
        ix[a] = jnp.where(c, ia, ib)
        ix[b] = jnp.where(c, ib, ia)

    acc = jnp.zeros((RB, K), jnp.float32)
    for k in range(K):
        m = jnp.min(vs[0], axis=1, keepdims=True)
        cand = jnp.where(vs[0] == m, ix[0], big)
        idx = jnp.min(cand, axis=1, keepdims=True)
        acc = jnp.where(kiota == k, idx, acc)
        cond = ix[0] == idx
        for j in range(G - 1):
            vs[j] = jnp.where(cond, vs[j + 1], vs[j])
            ix[j] = jnp.where(cond, ix[j + 1], ix[j])
        vs[G - 1] = jnp.where(cond, inf, vs[G - 1])
        ix[G - 1] = jnp.where(cond, big, ix[G - 1])
    out_ref[...] = acc.astype(jnp.int32) + offs


def _mlp_body(tab_ref, gat_ref,
              fd1_ref, fd1b_ref, fd2_ref, fd2b_ref,
              fg1_ref, fg1b_ref, fg2_ref, fg2b_ref,
              fc2_ref, fc2b_ref, res_ref, attn_ref):
    x = tab_ref[:, :D]
    zi = tab_ref[:, D:D + CP]
    kt = gat_ref[:, :D]
    kz = gat_ref[:, D:D + CP]
    z3 = jnp.broadcast_to(zi.reshape(RC, 1, CP), (RC, K, CP)).reshape(RC * K, CP)
    delta = z3 - kz
    pos = (
        jnp.dot(jnp.maximum(jnp.dot(delta, fd1_ref[...],
                                    preferred_element_type=jnp.float32)
                            + fd1b_ref[...], 0.0),
                fd2_ref[...], preferred_element_type=jnp.float32)
        + fd2b_ref[...]
    )
    x3 = jnp.broadcast_to(x.reshape(RC, 1, D), (RC, K, D)).reshape(RC * K, D)
    gin = x3 - kt + pos
    ap = (
        jnp.dot(jnp.maximum(jnp.dot(gin, fg1_ref[...],
                                    preferred_element_type=jnp.float32)
                            + fg1b_ref[...], 0.0),
                fg2_ref[...], preferred_element_type=jnp.float32)
        + fg2b_ref[...]
    )
    ap3 = (ap * jnp.float32(1.0 / math.sqrt(D))).reshape(RC, K, D)
    m = jnp.max(ap3, axis=1, keepdims=True)
    e = jnp.exp(ap3 - m)
    s = jnp.sum(e, axis=1, keepdims=True)
    attn3 = e / s
    v3 = (kt + pos).reshape(RC, K, D)
    red = jnp.sum(attn3 * v3, axis=1)
    res_ref[...] = (
        jnp.dot(red, fc2_ref[...], preferred_element_type=jnp.float32)
        + fc2b_ref[...] + x
    )
    attn_ref[...] = attn3.reshape(RC * K, D)


def _sc_gather_body(tab_hbm, idx_hbm, out_hbm,
                    idx0, idx1, buf0, buf1, gs0, gs1, ws0, ws1):
    wid = lax.axis_index("s") * NC + lax.axis_index("c")
    base = wid * RPW
    idxv, bufv, gsv, wsv = (idx0, idx1), (buf0, buf1), (gs0, gs1), (ws0, ws1)

    pltpu.sync_copy(idx_hbm.at[pl.ds(base, CH)], idx0)
    gathers = [pltpu.async_copy(tab_hbm.at[idx0], buf0, gs0), None]
    writes = [None, None]
    for i in range(NCH):
        p = i & 1
        q = 1 - p
        if i + 1 < NCH:
            pltpu.sync_copy(idx_hbm.at[pl.ds(base + (i + 1) * CH, CH)],
                            idxv[q])
            if writes[q] is not None:
                writes[q].wait()
            gathers[q] = pltpu.async_copy(tab_hbm.at[idxv[q]], bufv[q],
                                          gsv[q])
        gathers[p].wait()
        writes[p] = pltpu.async_copy(
            bufv[p], out_hbm.at[pl.ds(base + i * CH, CH)], wsv[p])
    writes[(NCH - 1) & 1].wait()
    if NCH > 1:
        writes[(NCH - 2) & 1].wait()


def kernel(features, xyz, fc1_W, fc1_b, fc2_W, fc2_b,
           fd1_W, fd1_b, fd2_W, fd2_b, fg1_W, fg1_b, fg2_W, fg2_b):
    f32 = jnp.float32
    feat2 = features.reshape(B * N, DP)
    xyzp = jnp.pad(xyz, ((0, 0), (0, 0), (0, CP - 3)))
    xyzt = jnp.transpose(xyzp, (0, 2, 1))
    fd1p = jnp.pad(fd1_W, ((0, CP - 3), (0, 0)))

    z_flat = xyzp.reshape(B * N, CP)
    table = pl.pallas_call(
        _prep_body,
        grid=(B * N // RP,),
        in_specs=[
            pl.BlockSpec((RP, DP), lambda i: (i, 0)),
            pl.BlockSpec((RP, CP), lambda i: (i, 0)),
            pl.BlockSpec((DP, D), lambda i: (0, 0)),
            pl.BlockSpec((1, D), lambda i: (0, 0)),
        ],
        out_specs=pl.BlockSpec((RP, W), lambda i: (i, 0)),
        out_shape=jax.ShapeDtypeStruct((B * N, W), f32),
    )(feat2, z_flat, fc1_W, fc1_b.reshape(1, D))

    mesh = plsc.VectorSubcoreMesh(core_axis_name="c", subcore_axis_name="s")
    gather = pl.kernel(
        _sc_gather_body,
        mesh=mesh,
        out_type=jax.ShapeDtypeStruct((SROWS, W), f32),
        scratch_types=[
            pltpu.VMEM((CH,), jnp.int32),
            pltpu.VMEM((CH,), jnp.int32),
            pltpu.VMEM((CH, W), f32),
            pltpu.VMEM((CH, W), f32),
            pltpu.SemaphoreType.DMA,
            pltpu.SemaphoreType.DMA,
            pltpu.SemaphoreType.DMA,
            pltpu.SemaphoreType.DMA,
        ],
    )

    def knn_seg(b, s):
        xyz_seg = xyzp[b, s * SN:(s + 1) * SN]
        return pl.pallas_call(
            functools.partial(_knn_body, offs=b * N),
            grid=(SN // RB,),
            in_specs=[
                pl.BlockSpec((RB, CP), lambda i: (i, 0)),
                pl.BlockSpec((CP, N), lambda i: (0, 0)),
            ],
            out_specs=pl.BlockSpec((RB, K), lambda i: (i, 0)),
            out_shape=jax.ShapeDtypeStruct((SN, K), jnp.int32),
        )(xyz_seg, xyzt[b])

    weights = (fd1p, fd1_b.reshape(1, D), fd2_W, fd2_b.reshape(1, D),
               fg1_W, fg1_b.reshape(1, D), fg2_W, fg2_b.reshape(1, D),
               fc2_W, fc2_b.reshape(1, D))

    def mlp_seg(tab_seg, gat_seg):
        return pl.pallas_call(
            _mlp_body,
            grid=(SN // RC,),
            in_specs=[
                pl.BlockSpec((RC, W), lambda i: (i, 0)),
                pl.BlockSpec((RC * K, W), lambda i: (i, 0)),
            ] + [pl.BlockSpec(w.shape, lambda i: (0, 0)) for w in weights],
            out_specs=[
                pl.BlockSpec((RC, D), lambda i: (i, 0)),
                pl.BlockSpec((RC * K, D), lambda i: (i, 0)),
            ],
            out_shape=[
                jax.ShapeDtypeStruct((SN, D), f32),
                jax.ShapeDtypeStruct((SROWS, D), f32),
            ],
        )(tab_seg, gat_seg, *weights)

    knns = [knn_seg(b, s) for b in range(B) for s in range(SPB)]
    gats = [gather(table, kn.reshape(SROWS)) for kn in knns]
    res_parts, attn_parts = [], []
    for g, (b, s) in zip(gats, [(b, s) for b in range(B) for s in range(SPB)]):
        lo = (b * SPB + s) * SN
        r, a = mlp_seg(lax.dynamic_slice_in_dim(table, lo, SN, 0), g)
        res_parts.append(r)
        attn_parts.append(a)

    res = jnp.concatenate(res_parts, axis=0).reshape(B, N, D)
    attn = jnp.concatenate(attn_parts, axis=0).reshape(B, N, K, D)
    return (res, attn)


# speedup vs baseline: 1.0882x; 1.0882x over previous
import functools
import math

import jax
import jax.numpy as jnp
from jax import lax
from jax.experimental import pallas as pl
from jax.experimental.pallas import tpu as pltpu
from jax.experimental.pallas import tpu_sc as plsc

B, N, K = 2, 4096, 16
DP, D = 64, 128
CP = 16
RB = 256
RC = 256
RP = 512
W = 256

SPB = 2
SEG = B * SPB
SN = N // SPB
SROWS = SN * K

NC, NS = 2, 16
NW = NC * NS
RPW = SROWS // NW
CH = 128
NCH = RPW // CH


def _prep_body(feat_ref, z_ref, w_ref, b_ref, out_ref):
    x = (
        jnp.dot(feat_ref[...], w_ref[...], preferred_element_type=jnp.float32)
        + b_ref[...]
    )
    out_ref[...] = jnp.concatenate(
        [x, z_ref[...], jnp.zeros((RP, W - D - CP), jnp.float32)], axis=1)


def _knn_body(xyzp_ref, xyzt_ref, out_ref, *, offs):
    blk = xyzp_ref[...]
    allt = xyzt_ref[...]
    sq_src = jnp.sum(blk * blk, axis=1, keepdims=True)
    sq_dst = jnp.sum(allt * allt, axis=0, keepdims=True)
    dot = jnp.dot(blk, allt, preferred_element_type=jnp.float32)
    dist = sq_src - 2.0 * dot + sq_dst
    fiota = lax.broadcasted_iota(jnp.int32, (RB, N), 1).astype(jnp.float32)
    kiota = lax.broadcasted_iota(jnp.int32, (RB, K), 1)
    big = jnp.float32(2.0 * N)
    inf = jnp.float32(jnp.inf)
    acc = jnp.zeros((RB, K), jnp.float32)
    for k in range(K):
        m = jnp.min(dist, axis=1, keepdims=True)
        cand = jnp.where(dist == m, fiota, big)
        idx = jnp.min(cand, axis=1, keepdims=True)
        acc = jnp.where(kiota == k, idx, acc)
        dist = jnp.where(fiota == idx, inf, dist)
    out_ref[...] = acc.astype(jnp.int32) + offs


def _mlp_body(tab_ref, gat_ref,
              fd1_ref, fd1b_ref, fd2_ref, fd2b_ref,
              fg1_ref, fg1b_ref, fg2_ref, fg2b_ref,
              fc2_ref, fc2b_ref, res_ref, attn_ref):
    x = tab_ref[:, :D]
    zi = tab_ref[:, D:D + CP]
    kt = gat_ref[:, :D]
    kz = gat_ref[:, D:D + CP]
    z3 = jnp.broadcast_to(zi.reshape(RC, 1, CP), (RC, K, CP)).reshape(RC * K, CP)
    delta = z3 - kz
    pos = (
        jnp.dot(jnp.maximum(jnp.dot(delta, fd1_ref[...],
                                    preferred_element_type=jnp.float32)
                            + fd1b_ref[...], 0.0),
                fd2_ref[...], preferred_element_type=jnp.float32)
        + fd2b_ref[...]
    )
    x3 = jnp.broadcast_to(x.reshape(RC, 1, D), (RC, K, D)).reshape(RC * K, D)
    gin = x3 - kt + pos
    ap = (
        jnp.dot(jnp.maximum(jnp.dot(gin, fg1_ref[...],
                                    preferred_element_type=jnp.float32)
                            + fg1b_ref[...], 0.0),
                fg2_ref[...], preferred_element_type=jnp.float32)
        + fg2b_ref[...]
    )
    ap3 = (ap * jnp.float32(1.0 / math.sqrt(D))).reshape(RC, K, D)
    m = jnp.max(ap3, axis=1, keepdims=True)
    e = jnp.exp(ap3 - m)
    s = jnp.sum(e, axis=1, keepdims=True)
    attn3 = e / s
    v3 = (kt + pos).reshape(RC, K, D)
    red = jnp.sum(attn3 * v3, axis=1)
    res_ref[...] = (
        jnp.dot(red, fc2_ref[...], preferred_element_type=jnp.float32)
        + fc2b_ref[...] + x
    )
    attn_ref[...] = attn3.reshape(RC * K, D)


def _sc_gather_body(tab_hbm, idx_hbm, out_hbm,
                    idx0, idx1, buf0, buf1, gs0, gs1, ws0, ws1):
    wid = lax.axis_index("s") * NC + lax.axis_index("c")
    base = wid * RPW
    idxv, bufv, gsv, wsv = (idx0, idx1), (buf0, buf1), (gs0, gs1), (ws0, ws1)

    pltpu.sync_copy(idx_hbm.at[pl.ds(base, CH)], idx0)
    gathers = [pltpu.async_copy(tab_hbm.at[idx0], buf0, gs0), None]
    writes = [None, None]
    for i in range(NCH):
        p = i & 1
        q = 1 - p
        if i + 1 < NCH:
            pltpu.sync_copy(idx_hbm.at[pl.ds(base + (i + 1) * CH, CH)],
                            idxv[q])
            if writes[q] is not None:
                writes[q].wait()
            gathers[q] = pltpu.async_copy(tab_hbm.at[idxv[q]], bufv[q],
                                          gsv[q])
        gathers[p].wait()
        writes[p] = pltpu.async_copy(
            bufv[p], out_hbm.at[pl.ds(base + i * CH, CH)], wsv[p])
    writes[(NCH - 1) & 1].wait()
    if NCH > 1:
        writes[(NCH - 2) & 1].wait()


def kernel(features, xyz, fc1_W, fc1_b, fc2_W, fc2_b,
           fd1_W, fd1_b, fd2_W, fd2_b, fg1_W, fg1_b, fg2_W, fg2_b):
    f32 = jnp.float32
    feat2 = features.reshape(B * N, DP)
    xyzp = jnp.pad(xyz, ((0, 0), (0, 0), (0, CP - 3)))
    xyzt = jnp.transpose(xyzp, (0, 2, 1))
    fd1p = jnp.pad(fd1_W, ((0, CP - 3), (0, 0)))

    z_flat = xyzp.reshape(B * N, CP)
    table = pl.pallas_call(
        _prep_body,
        grid=(B * N // RP,),
        in_specs=[
            pl.BlockSpec((RP, DP), lambda i: (i, 0)),
            pl.BlockSpec((RP, CP), lambda i: (i, 0)),
            pl.BlockSpec((DP, D), lambda i: (0, 0)),
            pl.BlockSpec((1, D), lambda i: (0, 0)),
        ],
        out_specs=pl.BlockSpec((RP, W), lambda i: (i, 0)),
        out_shape=jax.ShapeDtypeStruct((B * N, W), f32),
    )(feat2, z_flat, fc1_W, fc1_b.reshape(1, D))

    mesh = plsc.VectorSubcoreMesh(core_axis_name="c", subcore_axis_name="s")
    gather = pl.kernel(
        _sc_gather_body,
        mesh=mesh,
        out_type=jax.ShapeDtypeStruct((SROWS, W), f32),
        scratch_types=[
            pltpu.VMEM((CH,), jnp.int32),
            pltpu.VMEM((CH,), jnp.int32),
            pltpu.VMEM((CH, W), f32),
            pltpu.VMEM((CH, W), f32),
            pltpu.SemaphoreType.DMA,
            pltpu.SemaphoreType.DMA,
            pltpu.SemaphoreType.DMA,
            pltpu.SemaphoreType.DMA,
        ],
    )

    def knn_seg(b, s):
        xyz_seg = xyzp[b, s * SN:(s + 1) * SN]
        return pl.pallas_call(
            functools.partial(_knn_body, offs=b * N),
            grid=(SN // RB,),
            in_specs=[
                pl.BlockSpec((RB, CP), lambda i: (i, 0)),
                pl.BlockSpec((CP, N), lambda i: (0, 0)),
            ],
            out_specs=pl.BlockSpec((RB, K), lambda i: (i, 0)),
            out_shape=jax.ShapeDtypeStruct((SN, K), jnp.int32),
        )(xyz_seg, xyzt[b])

    weights = (fd1p, fd1_b.reshape(1, D), fd2_W, fd2_b.reshape(1, D),
               fg1_W, fg1_b.reshape(1, D), fg2_W, fg2_b.reshape(1, D),
               fc2_W, fc2_b.reshape(1, D))

    def mlp_seg(tab_seg, gat_seg):
        return pl.pallas_call(
            _mlp_body,
            grid=(SN // RC,),
            in_specs=[
                pl.BlockSpec((RC, W), lambda i: (i, 0)),
                pl.BlockSpec((RC * K, W), lambda i: (i, 0)),
            ] + [pl.BlockSpec(w.shape, lambda i: (0, 0)) for w in weights],
            out_specs=[
                pl.BlockSpec((RC, D), lambda i: (i, 0)),
                pl.BlockSpec((RC * K, D), lambda i: (i, 0)),
            ],
            out_shape=[
                jax.ShapeDtypeStruct((SN, D), f32),
                jax.ShapeDtypeStruct((SROWS, D), f32),
            ],
        )(tab_seg, gat_seg, *weights)

    knns = [knn_seg(b, s) for b in range(B) for s in range(SPB)]
    gats = [gather(table, kn.reshape(SROWS)) for kn in knns]
    res_parts, attn_parts = [], []
    for g, (b, s) in zip(gats, [(b, s) for b in range(B) for s in range(SPB)]):
        lo = (b * SPB + s) * SN
        r, a = mlp_seg(lax.dynamic_slice_in_dim(table, lo, SN, 0), g)
        res_parts.append(r)
        attn_parts.append(a)

    res = jnp.concatenate(res_parts, axis=0).reshape(B, N, D)
    attn = jnp.concatenate(attn_parts, axis=0).reshape(B, N, K, D)
    return (res, attn)


# speedup vs baseline: 1.1030x; 1.0136x over previous
import functools
import math

import jax
import jax.numpy as jnp
from jax import lax
from jax.experimental import pallas as pl
from jax.experimental.pallas import tpu as pltpu
from jax.experimental.pallas import tpu_sc as plsc

B, N, K = 2, 4096, 16
DP, D = 64, 128
CP = 16
RB = 256
RC = 256
RP = 512
W = 256

SPB = 1
SEG = B * SPB
SN = N // SPB
SROWS = SN * K

NC, NS = 2, 16
NW = NC * NS
RPW = SROWS // NW
CH = 128
NCH = RPW // CH


def _prep_body(feat_ref, z_ref, w_ref, b_ref, out_ref):
    x = (
        jnp.dot(feat_ref[...], w_ref[...], preferred_element_type=jnp.float32)
        + b_ref[...]
    )
    out_ref[...] = jnp.concatenate(
        [x, z_ref[...], jnp.zeros((RP, W - D - CP), jnp.float32)], axis=1)


def _knn_body(xyzp_ref, xyzt_ref, out_ref, *, offs):
    blk = xyzp_ref[...]
    allt = xyzt_ref[...]
    sq_src = jnp.sum(blk * blk, axis=1, keepdims=True)
    sq_dst = jnp.sum(allt * allt, axis=0, keepdims=True)
    dot = jnp.dot(blk, allt, preferred_element_type=jnp.float32)
    dist = sq_src - 2.0 * dot + sq_dst
    fiota = lax.broadcasted_iota(jnp.int32, (RB, N), 1).astype(jnp.float32)
    kiota = lax.broadcasted_iota(jnp.int32, (RB, K), 1)
    big = jnp.float32(2.0 * N)
    inf = jnp.float32(jnp.inf)
    acc = jnp.zeros((RB, K), jnp.float32)
    for k in range(K):
        m = jnp.min(dist, axis=1, keepdims=True)
        cand = jnp.where(dist == m, fiota, big)
        idx = jnp.min(cand, axis=1, keepdims=True)
        acc = jnp.where(kiota == k, idx, acc)
        dist = jnp.where(fiota == idx, inf, dist)
    out_ref[...] = acc.astype(jnp.int32) + offs


def _mlp_body(tab_ref, gat_ref,
              fd1_ref, fd1b_ref, fd2_ref, fd2b_ref,
              fg1_ref, fg1b_ref, fg2_ref, fg2b_ref,
              fc2_ref, fc2b_ref, res_ref, attn_ref):
    x = tab_ref[:, :D]
    zi = tab_ref[:, D:D + CP]
    kt = gat_ref[:, :D]
    kz = gat_ref[:, D:D + CP]
    z3 = jnp.broadcast_to(zi.reshape(RC, 1, CP), (RC, K, CP)).reshape(RC * K, CP)
    delta = z3 - kz
    pos = (
        jnp.dot(jnp.maximum(jnp.dot(delta, fd1_ref[...],
                                    preferred_element_type=jnp.float32)
                            + fd1b_ref[...], 0.0),
                fd2_ref[...], preferred_element_type=jnp.float32)
        + fd2b_ref[...]
    )
    x3 = jnp.broadcast_to(x.reshape(RC, 1, D), (RC, K, D)).reshape(RC * K, D)
    gin = x3 - kt + pos
    ap = (
        jnp.dot(jnp.maximum(jnp.dot(gin, fg1_ref[...],
                                    preferred_element_type=jnp.float32)
                            + fg1b_ref[...], 0.0),
                fg2_ref[...], preferred_element_type=jnp.float32)
        + fg2b_ref[...]
    )
    ap3 = (ap * jnp.float32(1.0 / math.sqrt(D))).reshape(RC, K, D)
    m = jnp.max(ap3, axis=1, keepdims=True)
    e = jnp.exp(ap3 - m)
    s = jnp.sum(e, axis=1, keepdims=True)
    attn3 = e / s
    v3 = (kt + pos).reshape(RC, K, D)
    red = jnp.sum(attn3 * v3, axis=1)
    res_ref[...] = (
        jnp.dot(red, fc2_ref[...], preferred_element_type=jnp.float32)
        + fc2b_ref[...] + x
    )
    attn_ref[...] = attn3.reshape(RC * K, D)


def _sc_gather_body(tab_hbm, idx_hbm, out_hbm,
                    idx0, idx1, buf0, buf1, gs0, gs1, ws0, ws1):
    wid = lax.axis_index("s") * NC + lax.axis_index("c")
    base = wid * RPW
    idxv, bufv, gsv, wsv = (idx0, idx1), (buf0, buf1), (gs0, gs1), (ws0, ws1)

    pltpu.sync_copy(idx_hbm.at[pl.ds(base, CH)], idx0)
    gathers = [pltpu.async_copy(tab_hbm.at[idx0], buf0, gs0), None]
    writes = [None, None]
    for i in range(NCH):
        p = i & 1
        q = 1 - p
        if i + 1 < NCH:
            pltpu.sync_copy(idx_hbm.at[pl.ds(base + (i + 1) * CH, CH)],
                            idxv[q])
            if writes[q] is not None:
                writes[q].wait()
            gathers[q] = pltpu.async_copy(tab_hbm.at[idxv[q]], bufv[q],
                                          gsv[q])
        gathers[p].wait()
        writes[p] = pltpu.async_copy(
            bufv[p], out_hbm.at[pl.ds(base + i * CH, CH)], wsv[p])
    writes[(NCH - 1) & 1].wait()
    if NCH > 1:
        writes[(NCH - 2) & 1].wait()


def kernel(features, xyz, fc1_W, fc1_b, fc2_W, fc2_b,
           fd1_W, fd1_b, fd2_W, fd2_b, fg1_W, fg1_b, fg2_W, fg2_b):
    f32 = jnp.float32
    feat2 = features.reshape(B * N, DP)
    xyzp = jnp.pad(xyz, ((0, 0), (0, 0), (0, CP - 3)))
    xyzt = jnp.transpose(xyzp, (0, 2, 1))
    fd1p = jnp.pad(fd1_W, ((0, CP - 3), (0, 0)))

    z_flat = xyzp.reshape(B * N, CP)
    table = pl.pallas_call(
        _prep_body,
        grid=(B * N // RP,),
        in_specs=[
            pl.BlockSpec((RP, DP), lambda i: (i, 0)),
            pl.BlockSpec((RP, CP), lambda i: (i, 0)),
            pl.BlockSpec((DP, D), lambda i: (0, 0)),
            pl.BlockSpec((1, D), lambda i: (0, 0)),
        ],
        out_specs=pl.BlockSpec((RP, W), lambda i: (i, 0)),
        out_shape=jax.ShapeDtypeStruct((B * N, W), f32),
    )(feat2, z_flat, fc1_W, fc1_b.reshape(1, D))

    mesh = plsc.VectorSubcoreMesh(core_axis_name="c", subcore_axis_name="s")
    gather = pl.kernel(
        _sc_gather_body,
        mesh=mesh,
        out_type=jax.ShapeDtypeStruct((SROWS, W), f32),
        scratch_types=[
            pltpu.VMEM((CH,), jnp.int32),
            pltpu.VMEM((CH,), jnp.int32),
            pltpu.VMEM((CH, W), f32),
            pltpu.VMEM((CH, W), f32),
            pltpu.SemaphoreType.DMA,
            pltpu.SemaphoreType.DMA,
            pltpu.SemaphoreType.DMA,
            pltpu.SemaphoreType.DMA,
        ],
    )

    def knn_seg(b, s):
        xyz_seg = xyzp[b, s * SN:(s + 1) * SN]
        return pl.pallas_call(
            functools.partial(_knn_body, offs=b * N),
            grid=(SN // RB,),
            in_specs=[
                pl.BlockSpec((RB, CP), lambda i: (i, 0)),
                pl.BlockSpec((CP, N), lambda i: (0, 0)),
            ],
            out_specs=pl.BlockSpec((RB, K), lambda i: (i, 0)),
            out_shape=jax.ShapeDtypeStruct((SN, K), jnp.int32),
        )(xyz_seg, xyzt[b])

    weights = (fd1p, fd1_b.reshape(1, D), fd2_W, fd2_b.reshape(1, D),
               fg1_W, fg1_b.reshape(1, D), fg2_W, fg2_b.reshape(1, D),
               fc2_W, fc2_b.reshape(1, D))

    def mlp_seg(tab_seg, gat_seg):
        return pl.pallas_call(
            _mlp_body,
            grid=(SN // RC,),
            in_specs=[
                pl.BlockSpec((RC, W), lambda i: (i, 0)),
                pl.BlockSpec((RC * K, W), lambda i: (i, 0)),
            ] + [pl.BlockSpec(w.shape, lambda i: (0, 0)) for w in weights],
            out_specs=[
                pl.BlockSpec((RC, D), lambda i: (i, 0)),
                pl.BlockSpec((RC * K, D), lambda i: (i, 0)),
            ],
            out_shape=[
                jax.ShapeDtypeStruct((SN, D), f32),
                jax.ShapeDtypeStruct((SROWS, D), f32),
            ],
        )(tab_seg, gat_seg, *weights)

    knns = [knn_seg(b, s) for b in range(B) for s in range(SPB)]
    gats = [gather(table, kn.reshape(SROWS)) for kn in knns]
    res_parts, attn_parts = [], []
    for g, (b, s) in zip(gats, [(b, s) for b in range(B) for s in range(SPB)]):
        lo = (b * SPB + s) * SN
        r, a = mlp_seg(lax.dynamic_slice_in_dim(table, lo, SN, 0), g)
        res_parts.append(r)
        attn_parts.append(a)

    res = jnp.concatenate(res_parts, axis=0).reshape(B, N, D)
    attn = jnp.concatenate(attn_parts, axis=0).reshape(B, N, K, D)
    return (res, attn)
